# Initial kernel scaffold; baseline (speedup 1.0000x reference)
import jax
import jax.numpy as jnp
from jax.experimental import pallas as pl


def kernel(a, rbf, distances, distance_vector, N, NM, f_dir, f_dynamics, r_dynamics, e_dynamics, W_rbf, b_rbf, W_a1, b_a1, W_a2, b_a2, W_f, W_fs1, b_fs1, W_fs2, b_fs2, W_r1, b_r1, W_r2, b_r2, W_re1, W_re2, W_e1, b_e1, W_e2, b_e2):
    raise NotImplementedError("write your pallas kernel here")


# speedup vs baseline: 5.2590x; 5.2590x over previous
import functools

import jax
import jax.numpy as jnp
from jax.experimental import pallas as pl

B, A, NN, NF, RES = 4, 256, 48, 128, 20
CUTOFF = 5.0
TA = 32
E = TA * NN
D3 = 3 * NF


def _prep_kernel(a_ref, W_a1, b_a1, W_a2, b_a2, W_fs1, b_fs1, W_fs2, b_fs2,
                 W_r1, b_r1, W_r2, b_r2, W_re1, W_re2, W_e1, b_e1, W_e2, b_e2,
                 am_out, W_fs_out, b_fs_out, W_r_out, b_r_out, W_re_out,
                 W_e_out, b_e_out):
    h = jnp.dot(a_ref[...], W_a1[...], preferred_element_type=jnp.float32) + b_a1[...]
    am_out[...] = jnp.dot(h, W_a2[...], preferred_element_type=jnp.float32) + b_a2[...]
    W_fs_out[...] = jnp.dot(W_fs1[...], W_fs2[...], preferred_element_type=jnp.float32)
    b_fs_out[...] = jnp.dot(b_fs1[...], W_fs2[...], preferred_element_type=jnp.float32) + b_fs2[...]
    W_r_out[...] = jnp.dot(W_r1[...], W_r2[...], preferred_element_type=jnp.float32)
    b_r_out[...] = jnp.dot(b_r1[...], W_r2[...], preferred_element_type=jnp.float32) + b_r2[...]
    W_re_out[...] = jnp.dot(W_re1[...], W_re2[...], preferred_element_type=jnp.float32)
    W_e_out[...] = jnp.dot(W_e1[...], W_e2[...], preferred_element_type=jnp.float32)
    b_e_out[...] = jnp.dot(b_e1[...], W_e2[...], preferred_element_type=jnp.float32) + b_e2[...]


def _main_kernel(a_ref, rbf_ref, dist_ref, dvec_ref, N_ref, NM_ref, fdir_ref,
                 fdyn_ref, rdyn_ref, am_ref, edyn_ref,
                 W_rbf, b_rbf, W_f, W_fs, b_fs, W_r, b_r, W_re, W_e, b_e,
                 a_out, fdir_out, fdyn_out, rdyn_out, e_out):
    i0 = pl.program_id(1) * TA

    rbf_ms = jnp.dot(rbf_ref[0], W_rbf[...], preferred_element_type=jnp.float32) + b_rbf[...]
    d = dist_ref[0]
    C = 0.5 * (jnp.cos(d * (jnp.pi / CUTOFF)) + 1.0) * (d < CUTOFF).astype(jnp.float32)
    rbf_ms = rbf_ms * C

    am_b = am_ref[0]
    r_b = rdyn_ref[0]
    oh = (N_ref[0] == jax.lax.broadcasted_iota(jnp.int32, (1, A), 1)).astype(jnp.float32)
    aj = jnp.dot(oh, am_b, preferred_element_type=jnp.float32)

    ai = am_ref[0, pl.ds(i0, TA), :]
    mij3 = (rbf_ms * aj).reshape(TA, NN, NF)
    msij3 = mij3 * ai[:, None, :]

    nm2 = NM_ref[0]
    nm3 = nm2.reshape(TA, NN, 1)
    a_sum = jnp.sum(msij3 * nm3, axis=1)

    msij = msij3.reshape(E, NF)
    fs = jnp.dot(msij, W_fs[...], preferred_element_type=jnp.float32) + b_fs[...]
    re = jnp.dot(msij, W_re[...], preferred_element_type=jnp.float32)
    fscore = jnp.dot(msij, W_f[...], preferred_element_type=jnp.float32)
    fm = fscore * nm2
    Fij = fm * dvec_ref[0]
    fdir_add = jnp.sum(Fij.reshape(TA, NN, 3), axis=1)

    G = jnp.dot(oh, r_b, preferred_element_type=jnp.float32)
    renm = re * nm2

    a_new = a_ref[0] + a_sum
    rvec = jnp.dot(a_new, W_r[...], preferred_element_type=jnp.float32) + b_r[...]
    evec = jnp.dot(a_new, W_e[...], preferred_element_type=jnp.float32) + b_e[...]

    r_old = rdyn_ref[0, pl.ds(i0, TA), :]
    de_acc = jnp.zeros((TA, NF), jnp.float32)
    for dd in range(3):
        sl = slice(dd * NF, (dd + 1) * NF)
        F_i_d = jnp.sum((fs * (fm * dvec_ref[0][:, dd:dd + 1])).reshape(TA, NN, NF), axis=1)
        dr_ext_d = jnp.sum((renm * G[:, sl]).reshape(TA, NN, NF), axis=1)
        f_new_d = fdyn_ref[0][:, sl] + F_i_d
        r_new_d = r_old[:, sl] + rvec * F_i_d + dr_ext_d
        fdyn_out[0, :, sl] = f_new_d
        rdyn_out[0, :, sl] = r_new_d
        de_acc = de_acc + f_new_d * r_new_d

    de_i = evec * (-de_acc)
    a_out[0] = a_new + de_i
    e_out[0] = edyn_ref[0] + de_i
    fdir_out[0] = fdir_ref[0] + fdir_add


@jax.jit
def kernel(a, rbf, distances, distance_vector, N, NM, f_dir, f_dynamics,
           r_dynamics, e_dynamics, W_rbf, b_rbf, W_a1, b_a1, W_a2, b_a2, W_f,
           W_fs1, b_fs1, W_fs2, b_fs2, W_r1, b_r1, W_r2, b_r2, W_re1, W_re2,
           W_e1, b_e1, W_e2, b_e2):
    f32 = jnp.float32
    row = lambda v: v.reshape(1, NF)

    am, W_fs, b_fs, W_r, b_r, W_re, W_e, b_e = pl.pallas_call(
        _prep_kernel,
        out_shape=[
            jax.ShapeDtypeStruct((B * A, NF), f32),
            jax.ShapeDtypeStruct((NF, NF), f32),
            jax.ShapeDtypeStruct((1, NF), f32),
            jax.ShapeDtypeStruct((NF, NF), f32),
            jax.ShapeDtypeStruct((1, NF), f32),
            jax.ShapeDtypeStruct((NF, NF), f32),
            jax.ShapeDtypeStruct((NF, NF), f32),
            jax.ShapeDtypeStruct((1, NF), f32),
        ],
    )(a.reshape(B * A, NF), W_a1, row(b_a1), W_a2, row(b_a2),
      W_fs1, row(b_fs1), W_fs2, row(b_fs2), W_r1, row(b_r1), W_r2, row(b_r2),
      W_re1, W_re2, W_e1, row(b_e1), W_e2, row(b_e2))

    am = am.reshape(B, A, NF)
    rdyn2 = r_dynamics.reshape(B, A, D3)
    fdyn2 = f_dynamics.reshape(B, A, D3)

    tile = lambda shape: pl.BlockSpec((1,) + shape, lambda b, i: (b, i, 0))
    table = lambda shape: pl.BlockSpec((1,) + shape, lambda b, i: (b, 0, 0))
    wspec = lambda w: pl.BlockSpec(w.shape, lambda b, i: (0,) * w.ndim)

    grid = (B, A // TA)
    a_o, fdir_o, fdyn_o, rdyn_o, e_o = pl.pallas_call(
        _main_kernel,
        grid=grid,
        in_specs=[
            tile((TA, NF)),
            tile((E, RES)),
            tile((E, 1)),
            tile((E, 3)),
            tile((E, 1)),
            tile((E, 1)),
            tile((TA, 3)),
            tile((TA, D3)),
            table((A, D3)),
            table((A, NF)),
            tile((TA, NF)),
            wspec(W_rbf), pl.BlockSpec((1, NF), lambda b, i: (0, 0)),
            wspec(W_f), wspec(W_fs1), pl.BlockSpec((1, NF), lambda b, i: (0, 0)),
            wspec(W_fs2), pl.BlockSpec((1, NF), lambda b, i: (0, 0)),
            wspec(W_re1), wspec(W_e1), pl.BlockSpec((1, NF), lambda b, i: (0, 0)),
        ],
        out_specs=[
            tile((TA, NF)),
            tile((TA, 3)),
            tile((TA, D3)),
            tile((TA, D3)),
            tile((TA, NF)),
        ],
        out_shape=[
            jax.ShapeDtypeStruct((B, A, NF), f32),
            jax.ShapeDtypeStruct((B, A, 3), f32),
            jax.ShapeDtypeStruct((B, A, D3), f32),
            jax.ShapeDtypeStruct((B, A, D3), f32),
            jax.ShapeDtypeStruct((B, A, NF), f32),
        ],
    )(a, rbf.reshape(B, A * NN, RES), distances.reshape(B, A * NN, 1),
      distance_vector.reshape(B, A * NN, 3), N.reshape(B, A * NN, 1).astype(jnp.int32),
      NM.reshape(B, A * NN, 1), f_dir, fdyn2, rdyn2, am, e_dynamics,
      W_rbf, b_rbf.reshape(1, NF), W_f, W_fs, b_fs, W_r, b_r, W_re, W_e, b_e)

    return (a_o, fdir_o, fdyn_o.reshape(B, A, 3, NF), rdyn_o.reshape(B, A, 3, NF), e_o)
